# baseline (speedup 1.0000x reference)
import functools

import jax
import jax.numpy as jnp
from jax import lax
from jax.experimental import pallas as pl
from jax.experimental.pallas import tpu as pltpu
from jax.experimental.pallas import tpu_sc as plsc

N = 10000
E = 160000
D = 256
HALF = 128

NUM_TILES = 16
EPT = E // NUM_TILES
SG = 80
EPT_PAD = SG * 128
CH = 40
NCH = SG // CH
STRIPE = 640
LAST_STRIPE = N - 15 * STRIPE


def _bcast_lane(vec, j):
    idx = jnp.full((16,), j, dtype=jnp.int32)
    return lax.gather(
        vec, idx[:, None],
        lax.GatherDimensionNumbers(
            offset_dims=(), collapsed_slice_dims=(0,), start_index_map=(0,)),
        slice_sizes=(1,),
        mode=lax.GatherScatterMode.PROMISE_IN_BOUNDS)


def _matmul_body(x_ref, w_ref, b_ref, o_ref):
    o_ref[0] = lax.dot_general(
        x_ref[...], w_ref[...], (((1,), (1,)), ((), ())),
        preferred_element_type=jnp.float32) + b_ref[0]


def _linear(x, W, b):
    R = 1000
    b2 = b.reshape(2, 1, HALF)
    return pl.pallas_call(
        _matmul_body,
        grid=(2, N // R),
        in_specs=[
            pl.BlockSpec((R, D), lambda c, i: (i, 0)),
            pl.BlockSpec((HALF, D), lambda c, i: (c, 0)),
            pl.BlockSpec((1, 1, HALF), lambda c, i: (c, 0, 0)),
        ],
        out_specs=pl.BlockSpec((1, R, HALF), lambda c, i: (c, i, 0)),
        out_shape=jax.ShapeDtypeStruct((2, N, HALF), jnp.float32),
    )(x, W, b2)


def _make_spmm():
    mesh = plsc.VectorSubcoreMesh(core_axis_name="c", subcore_axis_name="s")

    @functools.partial(
        pl.kernel, mesh=mesh,
        out_type=jax.ShapeDtypeStruct((N, D), jnp.float32),
        scratch_types=[
            pltpu.VMEM((CH, 128), jnp.int32),
            pltpu.VMEM((CH, 128), jnp.int32),
            pltpu.VMEM((CH, 128), jnp.float32),
            [pltpu.VMEM((128, HALF), jnp.float32) for _ in range(2)],
            [pltpu.SemaphoreType.DMA for _ in range(2)],
            [pltpu.SemaphoreType.DMA for _ in range(2)],
            pltpu.VMEM_SHARED((N, HALF), jnp.float32),
        ],
    )
    def spmm(h2_hbm, rows_hbm, cols_hbm, vals_hbm, out_hbm,
             rowbuf, idxbuf, valbuf, gbufs, gsems, ssems, acc):
        s = lax.axis_index("s")
        c = lax.axis_index("c")

        zvec = jnp.zeros((16,), jnp.float32)

        def z_body(i, _):
            for q in range(HALF // 16):
                gbufs[0][i, pl.ds(q * 16, 16)] = zvec
            return 0
        lax.fori_loop(0, 128, z_body, 0)
        row0 = pl.multiple_of(s * STRIPE, 8)

        @pl.when(s < 15)
        def _():
            for i in range(STRIPE // 128):
                pltpu.sync_copy(
                    gbufs[0],
                    acc.at[pl.ds(pl.multiple_of(row0 + i * 128, 8), 128)])

        @pl.when(s == 15)
        def _():
            for i in range(LAST_STRIPE // 128):
                pltpu.sync_copy(
                    gbufs[0],
                    acc.at[pl.ds(pl.multiple_of(row0 + i * 128, 8), 128)])
            rem0 = pl.multiple_of(row0 + (LAST_STRIPE // 128) * 128, 8)
            pltpu.sync_copy(gbufs[0].at[pl.ds(0, LAST_STRIPE % 128)],
                            acc.at[pl.ds(rem0, LAST_STRIPE % 128)])

        plsc.subcore_barrier()

        def issue_gather(k, b):
            pltpu.async_copy(h2_hbm.at[idxbuf.at[k]], gbufs[b], gsems[b])

        def wait_gather(b):
            pltpu.make_async_copy(h2_hbm.at[idxbuf.at[0]], gbufs[b],
                                  gsems[b]).wait()

        def issue_scatter(k, b):
            pltpu.async_copy(gbufs[b], acc.at[rowbuf.at[k]], ssems[b],
                             add=True)

        def wait_scatter(b):
            pltpu.make_async_copy(gbufs[b], acc.at[rowbuf.at[0]],
                                  ssems[b]).wait()

        def scale(k, b, lo, hi):
            def u_body(u, _):
                u16 = pl.multiple_of(u * 16, 8)
                vals16 = valbuf[k, pl.ds(u16, 16)]
                vjs = [_bcast_lane(vals16, j) for j in range(16)]
                for j in range(16):
                    r = u16 + j
                    for q in range(HALF // 16):
                        sl = pl.ds(q * 16, 16)
                        gbufs[b][r, sl] = gbufs[b][r, sl] * vjs[j]
                return 0
            lax.fori_loop(lo, hi, u_body, 0)

        off = c * N

        def chunk_body(c0, _):
            g0 = pl.multiple_of(c0 * CH, 8)
            pltpu.sync_copy(rows_hbm.at[s, pl.ds(g0, CH)], rowbuf)
            pltpu.sync_copy(cols_hbm.at[s, pl.ds(g0, CH)], idxbuf)
            pltpu.sync_copy(vals_hbm.at[s, pl.ds(g0, CH)], valbuf)

            def off_body(r, _):
                for q in range(HALF // 16):
                    sl = pl.ds(q * 16, 16)
                    idxbuf[r, sl] = idxbuf[r, sl] + off
                return 0
            lax.fori_loop(0, CH, off_body, 0)
            issue_gather(0, 0)

            def pair_body(p, _):
                for b in range(2):
                    kl = p * 2 + b
                    o = 1 - b
                    wait_gather(b)
                    scale(kl, b, 0, 1)

                    @pl.when(kl >= 1)
                    def _():
                        wait_scatter(o)

                    @pl.when(kl + 1 < CH)
                    def _():
                        issue_gather(kl + 1, o)

                    scale(kl, b, 1, 8)
                    issue_scatter(kl, b)
                return 0
            lax.fori_loop(0, CH // 2, pair_body, 0)
            wait_scatter(1)
            return 0
        lax.fori_loop(0, NCH, chunk_body, 0)
        plsc.subcore_barrier()

        col0 = pl.multiple_of(c * HALF, 128)

        @pl.when(s < 15)
        def _():
            pltpu.sync_copy(
                acc.at[pl.ds(row0, STRIPE)],
                out_hbm.at[pl.ds(row0, STRIPE), pl.ds(col0, HALF)])

        @pl.when(s == 15)
        def _():
            pltpu.sync_copy(
                acc.at[pl.ds(row0, LAST_STRIPE)],
                out_hbm.at[pl.ds(row0, LAST_STRIPE), pl.ds(col0, HALF)])

    return spmm


_spmm = _make_spmm()


def kernel(x, edge_index, edge_vals, W, b):
    h2 = _linear(x, W, b).reshape(2 * N, HALF)
    pad = EPT_PAD * NUM_TILES - E
    padidx = (jnp.arange(pad, dtype=jnp.int32) * 37) % N
    ei_pad = jnp.concatenate(
        [edge_index, jnp.broadcast_to(padidx, (2, pad))], axis=1)
    rows2 = ei_pad[0].reshape(NUM_TILES, SG, 128)
    cols2 = ei_pad[1].reshape(NUM_TILES, SG, 128)
    vals2 = jnp.pad(edge_vals, (0, pad)).reshape(NUM_TILES, SG, 128)
    return _spmm(h2, rows2, cols2, vals2)


# speedup vs baseline: 1.0329x; 1.0329x over previous
import functools

import jax
import jax.numpy as jnp
from jax import lax
from jax.experimental import pallas as pl
from jax.experimental.pallas import tpu as pltpu
from jax.experimental.pallas import tpu_sc as plsc

N = 10000
E = 160000
D = 256
HALF = 128

NUM_TILES = 16
EPT = E // NUM_TILES
SG = 80
EPT_PAD = SG * 128
CH = 40
NCH = SG // CH
STRIPE = 640
LAST_STRIPE = N - 15 * STRIPE


def _bcast_lane(vec, j):
    idx = jnp.full((16,), j, dtype=jnp.int32)
    return lax.gather(
        vec, idx[:, None],
        lax.GatherDimensionNumbers(
            offset_dims=(), collapsed_slice_dims=(0,), start_index_map=(0,)),
        slice_sizes=(1,),
        mode=lax.GatherScatterMode.PROMISE_IN_BOUNDS)


def _matmul_body(x_ref, w_ref, b_ref, o_ref):
    o_ref[0] = lax.dot_general(
        x_ref[...], w_ref[...], (((1,), (1,)), ((), ())),
        preferred_element_type=jnp.float32) + b_ref[0]


def _linear(x, W, b):
    R = 1000
    b2 = b.reshape(2, 1, HALF)
    return pl.pallas_call(
        _matmul_body,
        grid=(2, N // R),
        in_specs=[
            pl.BlockSpec((R, D), lambda c, i: (i, 0)),
            pl.BlockSpec((HALF, D), lambda c, i: (c, 0)),
            pl.BlockSpec((1, 1, HALF), lambda c, i: (c, 0, 0)),
        ],
        out_specs=pl.BlockSpec((1, R, HALF), lambda c, i: (c, i, 0)),
        out_shape=jax.ShapeDtypeStruct((2, N, HALF), jnp.float32),
    )(x, W, b2)


def _make_spmm():
    mesh = plsc.VectorSubcoreMesh(core_axis_name="c", subcore_axis_name="s")

    @functools.partial(
        pl.kernel, mesh=mesh,
        out_type=jax.ShapeDtypeStruct((N, D), jnp.float32),
        scratch_types=[
            pltpu.VMEM((CH, 128), jnp.int32),
            pltpu.VMEM((CH, 128), jnp.int32),
            pltpu.VMEM((CH, 128), jnp.float32),
            [pltpu.VMEM((128, HALF), jnp.float32) for _ in range(2)],
            [pltpu.SemaphoreType.DMA for _ in range(2)],
            [pltpu.SemaphoreType.DMA for _ in range(2)],
            pltpu.VMEM_SHARED((N, HALF), jnp.float32),
        ],
    )
    def spmm(h2_hbm, rows_hbm, cols_hbm, vals_hbm, out_hbm,
             rowbuf, idxbuf, valbuf, gbufs, gsems, ssems, acc):
        s = lax.axis_index("s")
        c = lax.axis_index("c")

        def stage_start(g0):
            pltpu.async_copy(rows_hbm.at[s, pl.ds(g0, CH)], rowbuf, gsems[0])
            pltpu.async_copy(cols_hbm.at[s, pl.ds(g0, CH)], idxbuf, gsems[0])
            pltpu.async_copy(vals_hbm.at[s, pl.ds(g0, CH)], valbuf, gsems[0])

        def stage_wait(g0):
            pltpu.make_async_copy(rows_hbm.at[s, pl.ds(g0, CH)], rowbuf,
                                  gsems[0]).wait()
            pltpu.make_async_copy(cols_hbm.at[s, pl.ds(g0, CH)], idxbuf,
                                  gsems[0]).wait()
            pltpu.make_async_copy(vals_hbm.at[s, pl.ds(g0, CH)], valbuf,
                                  gsems[0]).wait()

        stage_start(0)

        zvec = jnp.zeros((16,), jnp.float32)

        def z_body(i, _):
            for q in range(HALF // 16):
                gbufs[0][i, pl.ds(q * 16, 16)] = zvec
            return 0
        lax.fori_loop(0, 128, z_body, 0)
        row0 = pl.multiple_of(s * STRIPE, 8)

        def zdst(i):
            return acc.at[pl.ds(pl.multiple_of(row0 + i * 128, 8), 128)]

        @pl.when(s < 15)
        def _():
            for i in range(STRIPE // 128):
                pltpu.async_copy(gbufs[0], zdst(i), ssems[0])
            for i in range(STRIPE // 128):
                pltpu.make_async_copy(gbufs[0], zdst(i), ssems[0]).wait()

        @pl.when(s == 15)
        def _():
            for i in range(LAST_STRIPE // 128):
                pltpu.async_copy(gbufs[0], zdst(i), ssems[0])
            rem0 = pl.multiple_of(row0 + (LAST_STRIPE // 128) * 128, 8)
            rem = LAST_STRIPE % 128
            pltpu.async_copy(gbufs[0].at[pl.ds(0, rem)],
                             acc.at[pl.ds(rem0, rem)], ssems[0])
            for i in range(LAST_STRIPE // 128):
                pltpu.make_async_copy(gbufs[0], zdst(i), ssems[0]).wait()
            pltpu.make_async_copy(gbufs[0].at[pl.ds(0, rem)],
                                  acc.at[pl.ds(rem0, rem)], ssems[0]).wait()

        plsc.subcore_barrier()

        def issue_gather(k, b):
            pltpu.async_copy(h2_hbm.at[idxbuf.at[k]], gbufs[b], gsems[b])

        def wait_gather(b):
            pltpu.make_async_copy(h2_hbm.at[idxbuf.at[0]], gbufs[b],
                                  gsems[b]).wait()

        def issue_scatter(k, b):
            pltpu.async_copy(gbufs[b], acc.at[rowbuf.at[k]], ssems[b],
                             add=True)

        def wait_scatter(b):
            pltpu.make_async_copy(gbufs[b], acc.at[rowbuf.at[0]],
                                  ssems[b]).wait()

        def scale(k, b, lo, hi):
            def u_body(u, _):
                u16 = pl.multiple_of(u * 16, 8)
                vals16 = valbuf[k, pl.ds(u16, 16)]
                vjs = [_bcast_lane(vals16, j) for j in range(16)]
                for j in range(16):
                    r = u16 + j
                    for q in range(HALF // 16):
                        sl = pl.ds(q * 16, 16)
                        gbufs[b][r, sl] = gbufs[b][r, sl] * vjs[j]
                return 0
            lax.fori_loop(lo, hi, u_body, 0)

        off = c * N

        def chunk_body(c0, _):
            g0 = pl.multiple_of(c0 * CH, 8)

            @pl.when(c0 > 0)
            def _():
                stage_start(g0)
            stage_wait(g0)

            def off_body(r, _):
                for q in range(HALF // 16):
                    sl = pl.ds(q * 16, 16)
                    idxbuf[r, sl] = idxbuf[r, sl] + off
                return 0
            lax.fori_loop(0, CH, off_body, 0)
            issue_gather(0, 0)

            def pair_body(p, _):
                for b in range(2):
                    kl = p * 2 + b
                    o = 1 - b
                    wait_gather(b)

                    @pl.when(kl >= 1)
                    def _():
                        wait_scatter(o)

                    @pl.when(kl + 1 < CH)
                    def _():
                        issue_gather(kl + 1, o)

                    scale(kl, b, 0, 8)
                    issue_scatter(kl, b)
                return 0
            lax.fori_loop(0, CH // 2, pair_body, 0)
            wait_scatter(1)
            return 0
        lax.fori_loop(0, NCH, chunk_body, 0)
        plsc.subcore_barrier()

        col0 = pl.multiple_of(c * HALF, 128)

        @pl.when(s < 15)
        def _():
            pltpu.sync_copy(
                acc.at[pl.ds(row0, STRIPE)],
                out_hbm.at[pl.ds(row0, STRIPE), pl.ds(col0, HALF)])

        @pl.when(s == 15)
        def _():
            pltpu.sync_copy(
                acc.at[pl.ds(row0, LAST_STRIPE)],
                out_hbm.at[pl.ds(row0, LAST_STRIPE), pl.ds(col0, HALF)])

    return spmm


_spmm = _make_spmm()


def kernel(x, edge_index, edge_vals, W, b):
    h2 = _linear(x, W, b).reshape(2 * N, HALF)
    pad = EPT_PAD * NUM_TILES - E
    padidx = (jnp.arange(pad, dtype=jnp.int32) * 37) % N
    ei_pad = jnp.concatenate(
        [edge_index, jnp.broadcast_to(padidx, (2, pad))], axis=1)
    rows2 = ei_pad[0].reshape(NUM_TILES, SG, 128)
    cols2 = ei_pad[1].reshape(NUM_TILES, SG, 128)
    vals2 = jnp.pad(edge_vals, (0, pad)).reshape(NUM_TILES, SG, 128)
    return _spmm(h2, rows2, cols2, vals2)


# speedup vs baseline: 1.0363x; 1.0032x over previous
import functools

import jax
import jax.numpy as jnp
from jax import lax
from jax.experimental import pallas as pl
from jax.experimental.pallas import tpu as pltpu
from jax.experimental.pallas import tpu_sc as plsc

N = 10000
E = 160000
D = 256
HALF = 128

NUM_TILES = 16
EPT = E // NUM_TILES
SG = 80
EPT_PAD = SG * 128
CH = 40
NCH = SG // CH
STRIPE = 640
LAST_STRIPE = N - 15 * STRIPE


def _bcast_lane(vec, j):
    idx = jnp.full((16,), j, dtype=jnp.int32)
    return lax.gather(
        vec, idx[:, None],
        lax.GatherDimensionNumbers(
            offset_dims=(), collapsed_slice_dims=(0,), start_index_map=(0,)),
        slice_sizes=(1,),
        mode=lax.GatherScatterMode.PROMISE_IN_BOUNDS)


def _matmul_body(x_ref, w_ref, b_ref, o_ref):
    o_ref[0] = lax.dot_general(
        x_ref[...], w_ref[...], (((1,), (1,)), ((), ())),
        preferred_element_type=jnp.float32) + b_ref[0]


def _linear(x, W, b):
    R = 1000
    b2 = b.reshape(2, 1, HALF)
    return pl.pallas_call(
        _matmul_body,
        grid=(2, N // R),
        in_specs=[
            pl.BlockSpec((R, D), lambda c, i: (i, 0)),
            pl.BlockSpec((HALF, D), lambda c, i: (c, 0)),
            pl.BlockSpec((1, 1, HALF), lambda c, i: (c, 0, 0)),
        ],
        out_specs=pl.BlockSpec((1, R, HALF), lambda c, i: (c, i, 0)),
        out_shape=jax.ShapeDtypeStruct((2, N, HALF), jnp.float32),
    )(x, W, b2)


def _make_spmm():
    mesh = plsc.VectorSubcoreMesh(core_axis_name="c", subcore_axis_name="s")

    @functools.partial(
        pl.kernel, mesh=mesh,
        out_type=jax.ShapeDtypeStruct((N, D), jnp.float32),
        scratch_types=[
            pltpu.VMEM((CH, 128), jnp.int32),
            pltpu.VMEM((CH, 128), jnp.int32),
            pltpu.VMEM((CH, 128), jnp.float32),
            [pltpu.VMEM((128, HALF), jnp.float32) for _ in range(2)],
            [pltpu.SemaphoreType.DMA for _ in range(2)],
            [pltpu.SemaphoreType.DMA for _ in range(2)],
            pltpu.VMEM_SHARED((N, HALF), jnp.float32),
        ],
    )
    def spmm(h2_hbm, rows_hbm, cols_hbm, vals_hbm, out_hbm,
             rowbuf, idxbuf, valbuf, gbufs, gsems, ssems, acc):
        s = lax.axis_index("s")
        c = lax.axis_index("c")

        def stage_start(g0):
            pltpu.async_copy(rows_hbm.at[s, pl.ds(g0, CH)], rowbuf, gsems[0])
            pltpu.async_copy(cols_hbm.at[s, pl.ds(g0, CH)], idxbuf, gsems[0])
            pltpu.async_copy(vals_hbm.at[s, pl.ds(g0, CH)], valbuf, gsems[0])

        def stage_wait(g0):
            pltpu.make_async_copy(rows_hbm.at[s, pl.ds(g0, CH)], rowbuf,
                                  gsems[0]).wait()
            pltpu.make_async_copy(cols_hbm.at[s, pl.ds(g0, CH)], idxbuf,
                                  gsems[0]).wait()
            pltpu.make_async_copy(vals_hbm.at[s, pl.ds(g0, CH)], valbuf,
                                  gsems[0]).wait()

        stage_start(0)

        zvec = jnp.zeros((16,), jnp.float32)

        def z_body(i, _):
            for q in range(HALF // 16):
                gbufs[0][i, pl.ds(q * 16, 16)] = zvec
            return 0
        lax.fori_loop(0, 128, z_body, 0)
        row0 = pl.multiple_of(s * STRIPE, 8)

        def zdst(i):
            return acc.at[pl.ds(pl.multiple_of(row0 + i * 128, 8), 128)]

        @pl.when(s < 15)
        def _():
            for i in range(STRIPE // 128):
                pltpu.async_copy(gbufs[0], zdst(i), ssems[0])
            for i in range(STRIPE // 128):
                pltpu.make_async_copy(gbufs[0], zdst(i), ssems[0]).wait()

        @pl.when(s == 15)
        def _():
            for i in range(LAST_STRIPE // 128):
                pltpu.async_copy(gbufs[0], zdst(i), ssems[0])
            rem0 = pl.multiple_of(row0 + (LAST_STRIPE // 128) * 128, 8)
            rem = LAST_STRIPE % 128
            pltpu.async_copy(gbufs[0].at[pl.ds(0, rem)],
                             acc.at[pl.ds(rem0, rem)], ssems[0])
            for i in range(LAST_STRIPE // 128):
                pltpu.make_async_copy(gbufs[0], zdst(i), ssems[0]).wait()
            pltpu.make_async_copy(gbufs[0].at[pl.ds(0, rem)],
                                  acc.at[pl.ds(rem0, rem)], ssems[0]).wait()

        plsc.subcore_barrier()

        def issue_gather(k, b):
            pltpu.async_copy(h2_hbm.at[idxbuf.at[k]], gbufs[b], gsems[b])

        def wait_gather(b):
            pltpu.make_async_copy(h2_hbm.at[idxbuf.at[0]], gbufs[b],
                                  gsems[b]).wait()

        def issue_scatter(k, b):
            pltpu.async_copy(gbufs[b], acc.at[rowbuf.at[k]], ssems[b],
                             add=True)

        def wait_scatter(b):
            pltpu.make_async_copy(gbufs[b], acc.at[rowbuf.at[0]],
                                  ssems[b]).wait()

        def scale(k, b, lo, hi):
            @plsc.parallel_loop(lo, hi, unroll=2)
            def u_body(u):
                u16 = pl.multiple_of(u * 16, 8)
                vals16 = valbuf[k, pl.ds(u16, 16)]
                vjs = [_bcast_lane(vals16, j) for j in range(16)]
                for j in range(16):
                    r = u16 + j
                    for q in range(HALF // 16):
                        sl = pl.ds(q * 16, 16)
                        gbufs[b][r, sl] = gbufs[b][r, sl] * vjs[j]

        off = c * N

        def chunk_body(c0, _):
            g0 = pl.multiple_of(c0 * CH, 8)

            @pl.when(c0 > 0)
            def _():
                stage_start(g0)
            stage_wait(g0)

            def off_body(r, _):
                for q in range(HALF // 16):
                    sl = pl.ds(q * 16, 16)
                    idxbuf[r, sl] = idxbuf[r, sl] + off
                return 0
            lax.fori_loop(0, CH, off_body, 0)
            issue_gather(0, 0)

            def pair_body(p, _):
                for b in range(2):
                    kl = p * 2 + b
                    o = 1 - b
                    wait_gather(b)

                    @pl.when(kl >= 1)
                    def _():
                        wait_scatter(o)

                    @pl.when(kl + 1 < CH)
                    def _():
                        issue_gather(kl + 1, o)

                    scale(kl, b, 0, 8)
                    issue_scatter(kl, b)
                return 0
            lax.fori_loop(0, CH // 2, pair_body, 0)
            wait_scatter(1)
            return 0
        lax.fori_loop(0, NCH, chunk_body, 0)
        plsc.subcore_barrier()

        col0 = pl.multiple_of(c * HALF, 128)

        @pl.when(s < 15)
        def _():
            pltpu.sync_copy(
                acc.at[pl.ds(row0, STRIPE)],
                out_hbm.at[pl.ds(row0, STRIPE), pl.ds(col0, HALF)])

        @pl.when(s == 15)
        def _():
            pltpu.sync_copy(
                acc.at[pl.ds(row0, LAST_STRIPE)],
                out_hbm.at[pl.ds(row0, LAST_STRIPE), pl.ds(col0, HALF)])

    return spmm


_spmm = _make_spmm()


def kernel(x, edge_index, edge_vals, W, b):
    h2 = _linear(x, W, b).reshape(2 * N, HALF)
    pad = EPT_PAD * NUM_TILES - E
    padidx = (jnp.arange(pad, dtype=jnp.int32) * 37) % N
    ei_pad = jnp.concatenate(
        [edge_index, jnp.broadcast_to(padidx, (2, pad))], axis=1)
    rows2 = ei_pad[0].reshape(NUM_TILES, SG, 128)
    cols2 = ei_pad[1].reshape(NUM_TILES, SG, 128)
    vals2 = jnp.pad(edge_vals, (0, pad)).reshape(NUM_TILES, SG, 128)
    return _spmm(h2, rows2, cols2, vals2)
